# Initial kernel scaffold; baseline (speedup 1.0000x reference)
import jax
import jax.numpy as jnp
from jax.experimental import pallas as pl


def kernel(x, mask, W1, b1, g1, be1, W2, b2, W3, b3, g2, be2, W4, b4):
    raise NotImplementedError("write your pallas kernel here")


# speedup vs baseline: 1.3936x; 1.3936x over previous
import jax
import jax.numpy as jnp
from jax.experimental import pallas as pl
from jax.experimental.pallas import tpu as pltpu

_B, _M, _FEAT, _ENC = 16, 2048, 3, 256
_H1, _H2 = 128, 256
_N = _B * _M
_PHASES = 4


def _body(x_ref, m_ref, W1_ref, b1_ref, g1_ref, be1_ref, W2_ref, b2_ref,
          W3a_ref, W3b_ref, b3_ref, g2_ref, be2_ref, W4_ref, b4_ref,
          out_ref,
          h1p, hm, pooled, pp, cnt_v, sum1, sq1, scale1, shift1,
          sum2, sq2, scale2, shift2):
    s = pl.program_id(0)
    i = jax.lax.rem(s, _B)
    phase = jax.lax.div(s, _B)
    row = pl.ds(i * _M, _M)

    @pl.when(s == 0)
    def _init():
        cnt_v[...] = jnp.zeros_like(cnt_v)
        sum1[...] = jnp.zeros_like(sum1)
        sq1[...] = jnp.zeros_like(sq1)
        sum2[...] = jnp.zeros_like(sum2)
        sq2[...] = jnp.zeros_like(sq2)

    @pl.when(phase == 0)
    def _p1():
        xb = x_ref[...]
        m = m_ref[...]
        h = jnp.dot(xb, W1_ref[...], preferred_element_type=jnp.float32)
        h = h + b1_ref[...]
        h1p[row, :] = h
        hmask = h * m
        sum1[...] += jnp.sum(hmask, axis=0, keepdims=True)
        sq1[...] += jnp.sum(hmask * h, axis=0, keepdims=True)
        cnt_v[...] += jnp.sum(m)

    @pl.when(jnp.logical_and(phase == 1, i == 0))
    def _fin1():
        inv = 1.0 / cnt_v[:, :1]
        mean = sum1[...] * inv
        var = sq1[...] * inv - mean * mean
        sc = g1_ref[...] * jax.lax.rsqrt(var + 1e-5)
        scale1[...] = sc
        shift1[...] = be1_ref[...] - mean * sc

    @pl.when(phase == 1)
    def _p2():
        hp = h1p[row, :]
        hn = jnp.maximum(hp * scale1[...] + shift1[...], 0.0)
        hv = jnp.dot(hn, W2_ref[...], preferred_element_type=jnp.float32)
        hv = hv + b2_ref[...]
        m = m_ref[...]
        hm[row, :] = hv * m
        neg = jnp.where(m > 0.5, hv, -jnp.inf)
        pm = jnp.max(neg, axis=0, keepdims=True)
        nval = jnp.sum(m)
        clamp = jnp.where(nval < float(_M), 0.0, -jnp.inf)
        pooled[pl.ds(i, 1), :] = jnp.maximum(pm, clamp)

    @pl.when(jnp.logical_and(phase == 2, i == 0))
    def _pp():
        pp[...] = jnp.dot(pooled[...], W3b_ref[...],
                          preferred_element_type=jnp.float32) + b3_ref[...]

    @pl.when(phase == 2)
    def _p3():
        hv = hm[row, :]
        h2 = jnp.dot(hv, W3a_ref[...], preferred_element_type=jnp.float32)
        h2 = h2 + pp[pl.ds(i, 1), :]
        m = m_ref[...]
        h2m = h2 * m
        sum2[...] += jnp.sum(h2m, axis=0, keepdims=True)
        sq2[...] += jnp.sum(h2m * h2, axis=0, keepdims=True)

    @pl.when(jnp.logical_and(phase == 3, i == 0))
    def _fin2():
        inv = 1.0 / cnt_v[:, :1]
        mean = sum2[...] * inv
        var = sq2[...] * inv - mean * mean
        sc = g2_ref[...] * jax.lax.rsqrt(var + 1e-5)
        scale2[...] = sc
        shift2[...] = be2_ref[...] - mean * sc

    @pl.when(phase == 3)
    def _p4():
        hv = hm[row, :]
        h2 = jnp.dot(hv, W3a_ref[...], preferred_element_type=jnp.float32)
        h2 = h2 + pp[pl.ds(i, 1), :]
        h2n = jnp.maximum(h2 * scale2[...] + shift2[...], 0.0)
        o = jnp.dot(h2n, W4_ref[...], preferred_element_type=jnp.float32)
        o = o + b4_ref[...]
        m = m_ref[...]
        neg = jnp.where(m > 0.5, o, -jnp.inf)
        mx = jnp.max(neg, axis=0, keepdims=True)
        nval = jnp.sum(m)
        clamp = jnp.where(nval < float(_M), 0.0, -jnp.inf)
        out_ref[pl.ds(i, 1), :] = jnp.maximum(mx, clamp)


def kernel(x, mask, W1, b1, g1, be1, W2, b2, W3, b3, g2, be2, W4, b4):
    x2 = x.reshape(_N, _FEAT)
    mcol = mask.reshape(_N, 1).astype(jnp.float32)
    W3a = W3[:_H2]
    W3b = W3[_H2:]
    row_spec = pl.BlockSpec((_M, _FEAT), lambda s: (jax.lax.rem(s, _B), 0))
    m_spec = pl.BlockSpec((_M, 1), lambda s: (jax.lax.rem(s, _B), 0))

    def full(a):
        return pl.BlockSpec(a.shape, lambda s: (0,) * a.ndim)

    b1r, g1r, be1r = b1.reshape(1, _H1), g1.reshape(1, _H1), be1.reshape(1, _H1)
    b2r = b2.reshape(1, _H2)
    b3r, g2r, be2r = b3.reshape(1, _H2), g2.reshape(1, _H2), be2.reshape(1, _H2)
    b4r = b4.reshape(1, _ENC)
    ops = (x2, mcol, W1, b1r, g1r, be1r, W2, b2r, W3a, W3b, b3r, g2r, be2r,
           W4, b4r)
    in_specs = [row_spec, m_spec] + [full(a) for a in ops[2:]]

    out = pl.pallas_call(
        _body,
        grid=(_PHASES * _B,),
        in_specs=in_specs,
        out_specs=pl.BlockSpec((_B, _ENC), lambda s: (0, 0)),
        out_shape=jax.ShapeDtypeStruct((_B, _ENC), jnp.float32),
        scratch_shapes=[
            pltpu.VMEM((_N, _H1), jnp.float32),
            pltpu.VMEM((_N, _H2), jnp.float32),
            pltpu.VMEM((_B, _H2), jnp.float32),
            pltpu.VMEM((_B, _H2), jnp.float32),
            pltpu.VMEM((1, _H1), jnp.float32),
            pltpu.VMEM((1, _H1), jnp.float32),
            pltpu.VMEM((1, _H1), jnp.float32),
            pltpu.VMEM((1, _H1), jnp.float32),
            pltpu.VMEM((1, _H1), jnp.float32),
            pltpu.VMEM((1, _H2), jnp.float32),
            pltpu.VMEM((1, _H2), jnp.float32),
            pltpu.VMEM((1, _H2), jnp.float32),
            pltpu.VMEM((1, _H2), jnp.float32),
        ],
        compiler_params=pltpu.CompilerParams(
            vmem_limit_bytes=100 * 1024 * 1024,
        ),
    )(*ops)
    return out
